# baseline (speedup 1.0000x reference)
import jax
import jax.numpy as jnp
from jax import lax
from jax.experimental import pallas as pl
from jax.experimental.pallas import tpu as pltpu

_CHUNK = 15872
_FULL_CHUNKS = 63
_TAIL_OFF = _FULL_CHUNKS * _CHUNK
_TAIL = 1000000 - _TAIL_OFF
_NBUF = 2


def _dma_copy_body(src, dst, buf, tail_buf, in_sems, out_sems):
    def cols(j):
        return pl.ds(j * _CHUNK, _CHUNK)

    def step(j, _):
        b = j % _NBUF

        @pl.when(j < _FULL_CHUNKS)
        def _():
            @pl.when(j >= _NBUF)
            def _():
                pltpu.make_async_copy(buf.at[b], dst.at[:, cols(j - _NBUF)],
                                      out_sems.at[b]).wait()

            pltpu.make_async_copy(src.at[:, cols(j)], buf.at[b],
                                  in_sems.at[b]).start()

        @pl.when(j >= 1)
        def _():
            b1 = (j - 1) % _NBUF
            pltpu.make_async_copy(src.at[:, cols(j - 1)], buf.at[b1],
                                  in_sems.at[b1]).wait()
            pltpu.make_async_copy(buf.at[b1], dst.at[:, cols(j - 1)],
                                  out_sems.at[b1]).start()

        return _

    lax.fori_loop(0, _FULL_CHUNKS + 1, step, None)

    def drain(j, _):
        b = j % _NBUF
        pltpu.make_async_copy(buf.at[b], dst.at[:, cols(j)],
                              out_sems.at[b]).wait()
        return _

    lax.fori_loop(_FULL_CHUNKS - _NBUF, _FULL_CHUNKS, drain, None)

    tcols = pl.ds(_TAIL_OFF, _TAIL)
    cp_in = pltpu.make_async_copy(src.at[:, tcols], tail_buf, in_sems.at[0])
    cp_in.start()
    cp_in.wait()
    cp_out = pltpu.make_async_copy(tail_buf, dst.at[:, tcols], out_sems.at[0])
    cp_out.start()
    cp_out.wait()


def kernel(inputs, p_emb):
    n, d = p_emb.shape
    t = p_emb.T
    out = pl.pallas_call(
        _dma_copy_body,
        in_specs=[pl.BlockSpec(memory_space=pl.ANY)],
        out_specs=pl.BlockSpec(memory_space=pl.ANY),
        out_shape=jax.ShapeDtypeStruct(t.shape, t.dtype),
        scratch_shapes=[
            pltpu.VMEM((_NBUF, d, _CHUNK), jnp.float32),
            pltpu.VMEM((d, _TAIL), jnp.float32),
            pltpu.SemaphoreType.DMA((_NBUF,)),
            pltpu.SemaphoreType.DMA((_NBUF,)),
        ],
    )(t)
    return out.T


# speedup vs baseline: 1.1744x; 1.1744x over previous
import jax
from jax.experimental import pallas as pl

_BLOCK_COLS = 32768


def _copy_block(in_ref, out_ref):
    out_ref[...] = in_ref[...]


def kernel(inputs, p_emb):
    n, d = p_emb.shape
    t = p_emb.T
    grid = pl.cdiv(n, _BLOCK_COLS)
    out = pl.pallas_call(
        _copy_block,
        grid=(grid,),
        in_specs=[pl.BlockSpec((d, _BLOCK_COLS), lambda i: (0, i))],
        out_specs=pl.BlockSpec((d, _BLOCK_COLS), lambda i: (0, i)),
        out_shape=jax.ShapeDtypeStruct(t.shape, t.dtype),
    )(t)
    return out.T
